# baseline (speedup 1.0000x reference)
import functools

import jax
import jax.numpy as jnp
from jax import lax
from jax.experimental import pallas as pl
from jax.experimental.pallas import tpu as pltpu
from jax.experimental.pallas import tpu_sc as plsc

NC = 2
NS = 16
NW = NC * NS
CHUNK = 128


NBUF = 6
KLAG = 2


def _sc_gather_body(e0, e1, e2, i0, i1, i2, dep, g0, g1, g2, idx_v, rows_v, gsem, ssem):
    del dep
    n_per_w = i0.shape[0] // NW
    n_chunks = n_per_w // CHUNK
    n_grp = (n_chunks + NBUF - 1) // NBUF
    wid = lax.axis_index("s") * NC + lax.axis_index("c")
    base = wid * n_per_w
    for e, i, g in ((e0, i0, g0), (e1, i1, g1), (e2, i2, g2)):
        pltpu.sync_copy(i.at[pl.ds(base, n_per_w)], idx_v)
        for c in range(NBUF - KLAG):
            pltpu.async_copy(e.at[idx_v.at[pl.ds(c * CHUNK, CHUNK)]],
                             rows_v.at[c], gsem.at[c])

        def grp(go, carry, e=e, g=g):
            for b_ in range(NBUF):
                j = go * NBUF + b_
                jn = j + (NBUF - KLAG)
                bn = (b_ + NBUF - KLAG) % NBUF

                @pl.when(j < n_chunks)
                def _(g=g, j=j, b_=b_):
                    pltpu.make_async_copy(g.at[pl.ds(0, CHUNK)],
                                          rows_v.at[b_], gsem.at[b_]).wait()
                    pltpu.async_copy(rows_v.at[b_],
                                     g.at[pl.ds(base + j * CHUNK, CHUNK)],
                                     ssem.at[b_])

                @pl.when(jnp.logical_and(jn < n_chunks, j >= KLAG))
                def _(e=e, g=g, jn=jn, bn=bn):
                    pltpu.make_async_copy(g.at[pl.ds(0, CHUNK)],
                                          rows_v.at[bn], ssem.at[bn]).wait()
                    pltpu.async_copy(e.at[idx_v.at[pl.ds(jn * CHUNK, CHUNK)]],
                                     rows_v.at[bn], gsem.at[bn])

                @pl.when(jnp.logical_and(jn < n_chunks, j < KLAG))
                def _(e=e, jn=jn, bn=bn):
                    pltpu.async_copy(e.at[idx_v.at[pl.ds(jn * CHUNK, CHUNK)]],
                                     rows_v.at[bn], gsem.at[bn])
            return carry

        lax.fori_loop(0, n_grp, grp, 0)
        for b_ in range(NBUF):
            pltpu.make_async_copy(g.at[pl.ds(0, CHUNK)],
                                  rows_v.at[b_], ssem.at[b_]).wait()


def _tc_proj_body(g0b, g1b, g2b, intb, w_ref, ei_ref, b_ref, out_ref):
    w = w_ref[...]
    d = g0b.shape[1]
    t3 = jnp.dot(ei_ref[...], w[3 * d:4 * d, :],
                 preferred_element_type=jnp.float32)
    acc = jnp.dot(g0b[...], w[:d, :], preferred_element_type=jnp.float32)
    acc += jnp.dot(g1b[...], w[d:2 * d, :], preferred_element_type=jnp.float32)
    acc += jnp.dot(g2b[...], w[2 * d:3 * d, :], preferred_element_type=jnp.float32)
    it = intb[0, 0, :].reshape(intb.shape[2], 1)
    acc += jnp.where(it == 0, t3[0:1, :],
                     jnp.where(it == 1, t3[1:2, :], t3[2:3, :]))
    out_ref[...] = acc + b_ref[...]


def kernel(cat0, cat1, cat2, Interaction, E_cat0, E_cat1, E_cat2, E_inter, W, b):
    B, L = cat0.shape
    N = B * L
    D = E_cat0.shape[1]
    HD = W.shape[1]
    idx0 = cat0.reshape(N).astype(jnp.int32)
    idx1 = cat1.reshape(N).astype(jnp.int32)
    idx2 = cat2.reshape(N).astype(jnp.int32)

    TB = 4096
    halves = (3 * N // 5, 2 * N // 5)
    offs = (0, halves[0])
    mesh = plsc.VectorSubcoreMesh(core_axis_name="c", subcore_axis_name="s")

    def make_gath(nh):
        return pl.kernel(
            _sc_gather_body,
            out_type=[jax.ShapeDtypeStruct((nh, D), jnp.float32)] * 3,
            mesh=mesh,
            scratch_types=[
                pltpu.VMEM((nh // NW,), jnp.int32),
                pltpu.VMEM((NBUF, CHUNK, D), jnp.float32),
                pltpu.SemaphoreType.DMA((NBUF,)),
                pltpu.SemaphoreType.DMA((NBUF,)),
            ],
        )

    interb = Interaction.reshape(N).astype(jnp.int32)
    ei_pad = jnp.zeros((8, D), jnp.float32).at[:3].set(E_inter)
    b2 = b.reshape(1, HD)

    gs = []
    dep = b2
    for h, nh in enumerate(halves):
        o = offs[h]
        gs.append(make_gath(nh)(
            E_cat0, E_cat1, E_cat2,
            lax.slice(idx0, (o,), (o + nh,)),
            lax.slice(idx1, (o,), (o + nh,)),
            lax.slice(idx2, (o,), (o + nh,)), dep))
        dep = gs[-1][0]

    X = None
    for h, nh in enumerate(halves):
        nblk_h = nh // TB
        off = offs[h] // TB
        in_specs = [
            pl.BlockSpec((TB, D), lambda i: (i, 0)),
            pl.BlockSpec((TB, D), lambda i: (i, 0)),
            pl.BlockSpec((TB, D), lambda i: (i, 0)),
            pl.BlockSpec((1, 1, TB), lambda i, off=off: (i + off, 0, 0)),
            pl.BlockSpec((4 * D, HD), lambda i: (0, 0)),
            pl.BlockSpec((8, D), lambda i: (0, 0)),
            pl.BlockSpec((1, HD), lambda i: (0, 0)),
        ]
        args = list(gs[h]) + [interb.reshape(N // TB, 1, TB), W, ei_pad, b2]
        kwargs = {}
        if h > 0:
            in_specs.append(pl.BlockSpec(memory_space=pl.ANY))
            args.append(X)
            kwargs["input_output_aliases"] = {7: 0}

        def body(*refs):
            _tc_proj_body(*refs[:7], refs[-1])

        X = pl.pallas_call(
            body,
            grid=(nblk_h,),
            in_specs=in_specs,
            out_specs=pl.BlockSpec((TB, HD), lambda i, off=off: (i + off, 0)),
            out_shape=jax.ShapeDtypeStruct((N, HD), jnp.float32),
            compiler_params=pltpu.CompilerParams(
                dimension_semantics=("parallel",)),
            **kwargs,
        )(*args)

    return X.reshape(B, L, HD), B


# speedup vs baseline: 1.0148x; 1.0148x over previous
import functools

import jax
import jax.numpy as jnp
from jax import lax
from jax.experimental import pallas as pl
from jax.experimental.pallas import tpu as pltpu
from jax.experimental.pallas import tpu_sc as plsc

NC = 2
NS = 16
NW = NC * NS
CHUNK = 128


NBUF = 6
KLAG = 2


def _sc_gather_body(e0, e1, e2, i0, i1, i2, dep, g0, g1, g2, idx_v, rows_v, gsem, ssem):
    del dep
    n_per_w = i0.shape[0] // NW
    n_chunks = n_per_w // CHUNK
    n_grp = (n_chunks + NBUF - 1) // NBUF
    wid = lax.axis_index("s") * NC + lax.axis_index("c")
    base = wid * n_per_w
    for e, i, g in ((e0, i0, g0), (e1, i1, g1), (e2, i2, g2)):
        pltpu.sync_copy(i.at[pl.ds(base, n_per_w)], idx_v)
        for c in range(NBUF - KLAG):
            pltpu.async_copy(e.at[idx_v.at[pl.ds(c * CHUNK, CHUNK)]],
                             rows_v.at[c], gsem.at[c])

        def grp(go, carry, e=e, g=g):
            for b_ in range(NBUF):
                j = go * NBUF + b_
                jn = j + (NBUF - KLAG)
                bn = (b_ + NBUF - KLAG) % NBUF

                @pl.when(j < n_chunks)
                def _(g=g, j=j, b_=b_):
                    pltpu.make_async_copy(g.at[pl.ds(0, CHUNK)],
                                          rows_v.at[b_], gsem.at[b_]).wait()
                    pltpu.async_copy(rows_v.at[b_],
                                     g.at[pl.ds(base + j * CHUNK, CHUNK)],
                                     ssem.at[b_])

                @pl.when(jnp.logical_and(jn < n_chunks, j >= KLAG))
                def _(e=e, g=g, jn=jn, bn=bn):
                    pltpu.make_async_copy(g.at[pl.ds(0, CHUNK)],
                                          rows_v.at[bn], ssem.at[bn]).wait()
                    pltpu.async_copy(e.at[idx_v.at[pl.ds(jn * CHUNK, CHUNK)]],
                                     rows_v.at[bn], gsem.at[bn])

                @pl.when(jnp.logical_and(jn < n_chunks, j < KLAG))
                def _(e=e, jn=jn, bn=bn):
                    pltpu.async_copy(e.at[idx_v.at[pl.ds(jn * CHUNK, CHUNK)]],
                                     rows_v.at[bn], gsem.at[bn])
            return carry

        lax.fori_loop(0, n_grp, grp, 0)
        for b_ in range(NBUF):
            pltpu.make_async_copy(g.at[pl.ds(0, CHUNK)],
                                  rows_v.at[b_], ssem.at[b_]).wait()


def _tc_proj_body(g0b, g1b, g2b, intb, w_ref, ei_ref, b_ref, out_ref):
    w = w_ref[...]
    d = g0b.shape[1]
    t3 = jnp.dot(ei_ref[...], w[3 * d:4 * d, :],
                 preferred_element_type=jnp.float32)
    acc = jnp.dot(g0b[...], w[:d, :], preferred_element_type=jnp.float32)
    acc += jnp.dot(g1b[...], w[d:2 * d, :], preferred_element_type=jnp.float32)
    acc += jnp.dot(g2b[...], w[2 * d:3 * d, :], preferred_element_type=jnp.float32)
    it = intb[0, 0, :].reshape(intb.shape[2], 1)
    acc += jnp.where(it == 0, t3[0:1, :],
                     jnp.where(it == 1, t3[1:2, :], t3[2:3, :]))
    out_ref[...] = acc + b_ref[...]


def kernel(cat0, cat1, cat2, Interaction, E_cat0, E_cat1, E_cat2, E_inter, W, b):
    B, L = cat0.shape
    N = B * L
    D = E_cat0.shape[1]
    HD = W.shape[1]
    idx0 = cat0.reshape(N).astype(jnp.int32)
    idx1 = cat1.reshape(N).astype(jnp.int32)
    idx2 = cat2.reshape(N).astype(jnp.int32)

    TB = 6400
    halves = (N // 2, N // 2)
    offs = (0, halves[0])
    mesh = plsc.VectorSubcoreMesh(core_axis_name="c", subcore_axis_name="s")

    def make_gath(nh):
        return pl.kernel(
            _sc_gather_body,
            out_type=[jax.ShapeDtypeStruct((nh, D), jnp.float32)] * 3,
            mesh=mesh,
            scratch_types=[
                pltpu.VMEM((nh // NW,), jnp.int32),
                pltpu.VMEM((NBUF, CHUNK, D), jnp.float32),
                pltpu.SemaphoreType.DMA((NBUF,)),
                pltpu.SemaphoreType.DMA((NBUF,)),
            ],
        )

    interb = Interaction.reshape(N).astype(jnp.int32)
    ei_pad = jnp.zeros((8, D), jnp.float32).at[:3].set(E_inter)
    b2 = b.reshape(1, HD)

    gs = []
    dep = b2
    for h, nh in enumerate(halves):
        o = offs[h]
        gs.append(make_gath(nh)(
            E_cat0, E_cat1, E_cat2,
            lax.slice(idx0, (o,), (o + nh,)),
            lax.slice(idx1, (o,), (o + nh,)),
            lax.slice(idx2, (o,), (o + nh,)), dep))
        dep = gs[-1][0]

    X = None
    for h, nh in enumerate(halves):
        nblk_h = nh // TB
        off = offs[h] // TB
        in_specs = [
            pl.BlockSpec((TB, D), lambda i: (i, 0)),
            pl.BlockSpec((TB, D), lambda i: (i, 0)),
            pl.BlockSpec((TB, D), lambda i: (i, 0)),
            pl.BlockSpec((1, 1, TB), lambda i, off=off: (i + off, 0, 0)),
            pl.BlockSpec((4 * D, HD), lambda i: (0, 0)),
            pl.BlockSpec((8, D), lambda i: (0, 0)),
            pl.BlockSpec((1, HD), lambda i: (0, 0)),
        ]
        args = list(gs[h]) + [interb.reshape(N // TB, 1, TB), W, ei_pad, b2]
        kwargs = {}
        if h > 0:
            in_specs.append(pl.BlockSpec(memory_space=pl.ANY))
            args.append(X)
            kwargs["input_output_aliases"] = {7: 0}

        def body(*refs):
            _tc_proj_body(*refs[:7], refs[-1])

        X = pl.pallas_call(
            body,
            grid=(nblk_h,),
            in_specs=in_specs,
            out_specs=pl.BlockSpec((TB, HD), lambda i, off=off: (i + off, 0)),
            out_shape=jax.ShapeDtypeStruct((N, HD), jnp.float32),
            compiler_params=pltpu.CompilerParams(
                dimension_semantics=("parallel",)),
            **kwargs,
        )(*args)

    return X.reshape(B, L, HD), B


# speedup vs baseline: 1.0170x; 1.0021x over previous
import jax
import jax.numpy as jnp
from jax import lax
from jax.experimental import pallas as pl
from jax.experimental.pallas import tpu as pltpu
from jax.experimental.pallas import tpu_sc as plsc

NC = 2
NS = 16
NW = NC * NS
CHUNK = 128


NBUF = 6
KLAG = 2


def _sc_gather_body(e0, e1, e2, i0, i1, i2, dep, g0, g1, g2, idx_v, rows_v, gsem, ssem):
    del dep
    n_per_w = i0.shape[0] // NW
    n_chunks = n_per_w // CHUNK
    n_grp = (n_chunks + NBUF - 1) // NBUF
    wid = lax.axis_index("s") * NC + lax.axis_index("c")
    base = wid * n_per_w
    for e, i, g in ((e0, i0, g0), (e1, i1, g1), (e2, i2, g2)):
        pltpu.sync_copy(i.at[pl.ds(base, n_per_w)], idx_v)
        for c in range(NBUF - KLAG):
            pltpu.async_copy(e.at[idx_v.at[pl.ds(c * CHUNK, CHUNK)]],
                             rows_v.at[c], gsem.at[c])

        def grp(go, carry, e=e, g=g):
            for b_ in range(NBUF):
                j = go * NBUF + b_
                jn = j + (NBUF - KLAG)
                bn = (b_ + NBUF - KLAG) % NBUF

                @pl.when(j < n_chunks)
                def _(g=g, j=j, b_=b_):
                    pltpu.make_async_copy(g.at[pl.ds(0, CHUNK)],
                                          rows_v.at[b_], gsem.at[b_]).wait()
                    pltpu.async_copy(rows_v.at[b_],
                                     g.at[pl.ds(base + j * CHUNK, CHUNK)],
                                     ssem.at[b_])

                @pl.when(jnp.logical_and(jn < n_chunks, j >= KLAG))
                def _(e=e, g=g, jn=jn, bn=bn):
                    pltpu.make_async_copy(g.at[pl.ds(0, CHUNK)],
                                          rows_v.at[bn], ssem.at[bn]).wait()
                    pltpu.async_copy(e.at[idx_v.at[pl.ds(jn * CHUNK, CHUNK)]],
                                     rows_v.at[bn], gsem.at[bn])

                @pl.when(jnp.logical_and(jn < n_chunks, j < KLAG))
                def _(e=e, jn=jn, bn=bn):
                    pltpu.async_copy(e.at[idx_v.at[pl.ds(jn * CHUNK, CHUNK)]],
                                     rows_v.at[bn], gsem.at[bn])
            return carry

        lax.fori_loop(0, n_grp, grp, 0)
        for b_ in range(NBUF):
            pltpu.make_async_copy(g.at[pl.ds(0, CHUNK)],
                                  rows_v.at[b_], ssem.at[b_]).wait()


def _tc_proj_body(g0b, g1b, g2b, intb, w_ref, ei_ref, b_ref, out_ref):
    w = w_ref[...]
    d = g0b.shape[1]
    t3 = jnp.dot(ei_ref[...], w[3 * d:4 * d, :],
                 preferred_element_type=jnp.float32)
    acc = jnp.dot(g0b[...], w[:d, :], preferred_element_type=jnp.float32)
    acc += jnp.dot(g1b[...], w[d:2 * d, :], preferred_element_type=jnp.float32)
    acc += jnp.dot(g2b[...], w[2 * d:3 * d, :], preferred_element_type=jnp.float32)
    it = intb[0, 0, :].reshape(intb.shape[2], 1)
    acc += jnp.where(it == 0, t3[0:1, :],
                     jnp.where(it == 1, t3[1:2, :], t3[2:3, :]))
    out_ref[...] = acc + b_ref[...]


def kernel(cat0, cat1, cat2, Interaction, E_cat0, E_cat1, E_cat2, E_inter, W, b):
    B, L = cat0.shape
    N = B * L
    D = E_cat0.shape[1]
    HD = W.shape[1]
    idx0 = cat0.reshape(N).astype(jnp.int32)
    idx1 = cat1.reshape(N).astype(jnp.int32)
    idx2 = cat2.reshape(N).astype(jnp.int32)

    TB = 6400
    halves = (N // 2, N // 2)
    offs = (0, halves[0])
    mesh = plsc.VectorSubcoreMesh(core_axis_name="c", subcore_axis_name="s")

    def make_gath(nh):
        return pl.kernel(
            _sc_gather_body,
            out_type=[jax.ShapeDtypeStruct((nh, D), jnp.float32)] * 3,
            mesh=mesh,
            scratch_types=[
                pltpu.VMEM((nh // NW,), jnp.int32),
                pltpu.VMEM((NBUF, CHUNK, D), jnp.float32),
                pltpu.SemaphoreType.DMA((NBUF,)),
                pltpu.SemaphoreType.DMA((NBUF,)),
            ],
        )

    interb = Interaction.reshape(N).astype(jnp.int32)
    ei_pad = jnp.zeros((8, D), jnp.float32).at[:3].set(E_inter)
    b2 = b.reshape(1, HD)

    gs = []
    dep = b2
    for h, nh in enumerate(halves):
        o = offs[h]
        gs.append(make_gath(nh)(
            E_cat0, E_cat1, E_cat2,
            lax.slice(idx0, (o,), (o + nh,)),
            lax.slice(idx1, (o,), (o + nh,)),
            lax.slice(idx2, (o,), (o + nh,)), dep))
        dep = gs[-1][0]

    X = None
    for h, nh in enumerate(halves):
        nblk_h = nh // TB
        off = offs[h] // TB
        in_specs = [
            pl.BlockSpec((TB, D), lambda i: (i, 0)),
            pl.BlockSpec((TB, D), lambda i: (i, 0)),
            pl.BlockSpec((TB, D), lambda i: (i, 0)),
            pl.BlockSpec((1, 1, TB), lambda i, off=off: (i + off, 0, 0)),
            pl.BlockSpec((4 * D, HD), lambda i: (0, 0)),
            pl.BlockSpec((8, D), lambda i: (0, 0)),
            pl.BlockSpec((1, HD), lambda i: (0, 0)),
        ]
        args = list(gs[h]) + [interb.reshape(N // TB, 1, TB), W, ei_pad, b2]
        kwargs = {}
        if h > 0:
            in_specs.append(pl.BlockSpec(memory_space=pl.ANY))
            args.append(X)
            kwargs["input_output_aliases"] = {7: 0}

        def body(*refs):
            _tc_proj_body(*refs[:7], refs[-1])

        X = pl.pallas_call(
            body,
            grid=(nblk_h,),
            in_specs=in_specs,
            out_specs=pl.BlockSpec((TB, HD), lambda i, off=off: (i + off, 0)),
            out_shape=jax.ShapeDtypeStruct((N, HD), jnp.float32),
            compiler_params=pltpu.CompilerParams(
                dimension_semantics=("parallel",)),
            **kwargs,
        )(*args)

    return X.reshape(B, L, HD), B
